# baseline (speedup 1.0000x reference)
import functools

import jax
import jax.numpy as jnp
from jax import lax
from jax.experimental import pallas as pl
from jax.experimental.pallas import tpu as pltpu
from jax.experimental.pallas import tpu_sc as plsc

_V = 1000000
_H = 20
_BT = 16384
_D = 64
_NC = 2
_NS = 16
_NW = _NC * _NS
_C = 256
_NB = _C // 128
_NCH = _BT // (_NW * _C)
_NT = _D // 8
_CP = 129
_FB = 4096

_mesh = plsc.VectorSubcoreMesh(core_axis_name="c", subcore_axis_name="s")

_DNUMS = lax.GatherDimensionNumbers(
    offset_dims=(), collapsed_slice_dims=(0,), start_index_map=(0,)
)


def _fmt_body(wt_ref, o_ref):
    for m in range(_FB // 256):
        a = wt_ref[:, pl.ds(m * 256, 128)].T
        b = wt_ref[:, pl.ds(m * 256 + 128, 128)].T
        o_ref[pl.ds(m * 128, 128), 0:_D] = a
        o_ref[pl.ds(m * 128, 128), _D : 2 * _D] = b


_NG = (_V + _FB - 1) // _FB
_VP = _NG * (_FB // 2)

_fmt = pl.pallas_call(
    _fmt_body,
    grid=(_NG,),
    in_specs=[pl.BlockSpec((_D, _FB), lambda i: (0, i))],
    out_specs=pl.BlockSpec((_FB // 2, 2 * _D), lambda i: (i, 0)),
    out_shape=jax.ShapeDtypeStruct((_VP, 2 * _D), jnp.float32),
)


@functools.partial(
    pl.kernel,
    out_type=jax.ShapeDtypeStruct((_H, _NT, _BT // 128, 8, 128), jnp.float32),
    mesh=_mesh,
    scratch_types=[
        pltpu.VMEM((2, _C), jnp.int32),
        pltpu.VMEM((2, _C), jnp.int32),
        pltpu.VMEM((2, _C), jnp.int32),
        pltpu.VMEM((2, _C, 2 * _D), jnp.float32),
        pltpu.VMEM((2, _NB, _NT, 8, _CP), jnp.float32),
        pltpu.SemaphoreType.DMA,
        pltpu.SemaphoreType.DMA,
    ],
    compiler_params=pltpu.CompilerParams(
        use_tc_tiling_on_sc=False, needs_layout_passes=False
    ),
)
def _emb_kernel(
    idx_hbm, table_hbm, out_hbm, idx_v, row_v, par_v, rows_v, out_v, gsem, osem
):
    wid = lax.axis_index("s") * _NC + lax.axis_index("c")
    lane = lax.iota(jnp.int32, 16)
    lane_t = lane // 8
    lane_r = lax.rem(lane, 8)
    ngch = _H * _NCH

    def chunk_off(g):
        h = g // _NCH
        sub = lax.rem(g, _NCH)
        return h * _BT + wid * (_NCH * _C) + sub * _C

    def idx_stage(g, slot):
        pltpu.sync_copy(idx_hbm.at[pl.ds(chunk_off(g), _C)], idx_v.at[slot])

        def split(k, _):
            v = idx_v[slot, pl.ds(k * 16, 16)]
            row_v[slot, pl.ds(k * 16, 16)] = ((v >> 8) << 7) | (v & 127)
            par_v[slot, pl.ds(k * 16, 16)] = ((v >> 7) & 1) * _D
            return 0

        lax.fori_loop(0, _C // 16, split, 0)

    def gather(slot):
        return pltpu.async_copy(
            table_hbm.at[row_v.at[slot]], rows_v.at[slot], gsem
        )

    def out_copy(g, t):
        h = g // _NCH
        sub = lax.rem(g, _NCH)
        slot = lax.rem(g, 2)
        blk0 = wid * (_NCH * _NB) + sub * _NB
        return pltpu.make_async_copy(
            out_v.at[slot, :, t, :, pl.ds(0, 128)],
            out_hbm.at[h, t, pl.ds(blk0, _NB), :, :],
            osem,
        )

    idx_stage(0, 0)
    gather(0)

    def chunk(g, _):
        slot = lax.rem(g, 2)
        nslot = 1 - slot

        @pl.when(g < ngch - 1)
        def _prefetch():
            idx_stage(g + 1, nslot)
            gather(nslot)

        pltpu.make_async_copy(
            table_hbm.at[row_v.at[slot]], rows_v.at[slot], gsem
        ).wait()

        @pl.when(g > 1)
        def _drain():
            for t in range(_NT):
                out_copy(g - 2, t).wait()

        slot_splat = jnp.full((16,), slot, jnp.int32)

        def b_body(bb, _):
            base_b = jnp.full((16,), bb, jnp.int32)

            def cg_body(cg, _):
                c0 = cg * 16
                parv16 = par_v[slot, pl.ds(bb * 128 + c0, 16)]

                def k_body(k, _):
                    j = bb * 128 + c0 + k
                    par_b = lax.gather(
                        parv16,
                        jnp.full((16, 1), k, jnp.int32),
                        _DNUMS,
                        slice_sizes=(1,),
                        mode=lax.GatherScatterMode.PROMISE_IN_BOUNDS,
                    )
                    j_splat = jnp.full((16,), j, jnp.int32)
                    base_c = jnp.full((16,), c0 + k, jnp.int32)
                    for d0 in range(0, _D, 16):
                        val = plsc.load_gather(
                            rows_v, [slot_splat, j_splat, par_b + (d0 + lane)]
                        )
                        plsc.store_scatter(
                            out_v,
                            [slot_splat, base_b, lane_t + (d0 // 8), lane_r, base_c],
                            val,
                        )
                    return 0

                lax.fori_loop(0, 16, k_body, 0)
                return 0

            lax.fori_loop(0, 8, cg_body, 0)
            return 0

        lax.fori_loop(0, _NB, b_body, 0)

        for t in range(_NT):
            out_copy(g, t).start()
        return 0

    lax.fori_loop(0, ngch, chunk, 0)
    for t in range(_NT):
        out_copy(ngch - 2, t).wait()
        out_copy(ngch - 1, t).wait()


def kernel(indices, weight):
    idx_t = indices.astype(jnp.int32).T.reshape(-1)
    table = _fmt(weight.T)
    p5 = _emb_kernel(idx_t, table)
    return p5.transpose((2, 4, 0, 1, 3)).reshape(_BT, _H, _D)


# speedup vs baseline: 1.0340x; 1.0340x over previous
import functools

import jax
import jax.numpy as jnp
from jax import lax
from jax.experimental import pallas as pl
from jax.experimental.pallas import tpu as pltpu
from jax.experimental.pallas import tpu_sc as plsc

_V = 1000000
_H = 20
_BT = 16384
_D = 64
_NC = 2
_NS = 16
_NW = _NC * _NS
_C = 256
_NB = _C // 128
_NCH = _BT // (_NW * _C)
_NT = _D // 8
_CP = 129
_FB = 4096

_mesh = plsc.VectorSubcoreMesh(core_axis_name="c", subcore_axis_name="s")

_DNUMS = lax.GatherDimensionNumbers(
    offset_dims=(), collapsed_slice_dims=(0,), start_index_map=(0,)
)


def _fmt_body(wt_ref, o_ref):
    for m in range(_FB // 256):
        a = wt_ref[:, pl.ds(m * 256, 128)].T
        b = wt_ref[:, pl.ds(m * 256 + 128, 128)].T
        o_ref[pl.ds(m * 128, 128), 0:_D] = a
        o_ref[pl.ds(m * 128, 128), _D : 2 * _D] = b


_NG = (_V + _FB - 1) // _FB
_VP = _NG * (_FB // 2)

_fmt = pl.pallas_call(
    _fmt_body,
    grid=(_NG,),
    in_specs=[pl.BlockSpec((_D, _FB), lambda i: (0, i))],
    out_specs=pl.BlockSpec((_FB // 2, 2 * _D), lambda i: (i, 0)),
    out_shape=jax.ShapeDtypeStruct((_VP, 2 * _D), jnp.float32),
)


@functools.partial(
    pl.kernel,
    out_type=jax.ShapeDtypeStruct((_H, _NT, _BT // 128, 8, 128), jnp.float32),
    mesh=_mesh,
    scratch_types=[
        pltpu.VMEM((2, _C), jnp.int32),
        pltpu.VMEM((2, _C), jnp.int32),
        pltpu.VMEM((2, _C, _D), jnp.float32),
        pltpu.VMEM((2, _NB, _NT, 8, _CP), jnp.float32),
        pltpu.SemaphoreType.DMA,
        pltpu.SemaphoreType.DMA,
    ],
    compiler_params=pltpu.CompilerParams(
        use_tc_tiling_on_sc=False, needs_layout_passes=False
    ),
)
def _emb_kernel(
    idx_hbm, table_hbm, out_hbm, idx_v, row_v, rows_v, out_v, gsem, osem
):
    wid = lax.axis_index("s") * _NC + lax.axis_index("c")
    lane = lax.iota(jnp.int32, 16)
    lane_t = lane // 8
    lane_r = lax.rem(lane, 8)
    ngch = _H * _NCH

    def chunk_off(g):
        h = g // _NCH
        sub = lax.rem(g, _NCH)
        return h * _BT + wid * (_NCH * _C) + sub * _C

    def idx_stage(g, slot):
        pltpu.sync_copy(idx_hbm.at[pl.ds(chunk_off(g), _C)], idx_v.at[slot])

        def split(k, _):
            v = idx_v[slot, pl.ds(k * 16, 16)]
            row_v[slot, pl.ds(k * 16, 16)] = (
                ((v >> 8) << 8) | ((v & 127) << 1) | ((v >> 7) & 1)
            )
            return 0

        lax.fori_loop(0, _C // 16, split, 0)

    def gather(slot):
        return pltpu.async_copy(
            table_hbm.at[row_v.at[slot]], rows_v.at[slot], gsem
        )

    def out_copy(g, t):
        h = g // _NCH
        sub = lax.rem(g, _NCH)
        slot = lax.rem(g, 2)
        blk0 = wid * (_NCH * _NB) + sub * _NB
        return pltpu.make_async_copy(
            out_v.at[slot, :, t, :, pl.ds(0, 128)],
            out_hbm.at[h, t, pl.ds(blk0, _NB), :, :],
            osem,
        )

    idx_stage(0, 0)
    gather(0)

    def chunk(g, _):
        slot = lax.rem(g, 2)
        nslot = 1 - slot

        @pl.when(g < ngch - 1)
        def _prefetch():
            idx_stage(g + 1, nslot)
            gather(nslot)

        pltpu.make_async_copy(
            table_hbm.at[row_v.at[slot]], rows_v.at[slot], gsem
        ).wait()

        @pl.when(g > 1)
        def _drain():
            for t in range(_NT):
                out_copy(g - 2, t).wait()

        slot_splat = jnp.full((16,), slot, jnp.int32)

        def b_body(bb, _):
            base_b = jnp.full((16,), bb, jnp.int32)

            def c_body(c, _):
                j = bb * 128 + c
                base_c = jnp.full((16,), c, jnp.int32)
                for d0 in range(0, _D, 16):
                    val = rows_v[slot, j, pl.ds(d0, 16)]
                    plsc.store_scatter(
                        out_v,
                        [slot_splat, base_b, lane_t + (d0 // 8), lane_r, base_c],
                        val,
                    )
                return 0

            lax.fori_loop(0, 128, c_body, 0)
            return 0

        lax.fori_loop(0, _NB, b_body, 0)

        for t in range(_NT):
            out_copy(g, t).start()
        return 0

    lax.fori_loop(0, ngch, chunk, 0)
    for t in range(_NT):
        out_copy(ngch - 2, t).wait()
        out_copy(ngch - 1, t).wait()


def kernel(indices, weight):
    idx_t = indices.astype(jnp.int32).T.reshape(-1)
    table = _fmt(weight.T).reshape(2 * _VP, _D)
    p5 = _emb_kernel(idx_t, table)
    return p5.transpose((2, 4, 0, 1, 3)).reshape(_BT, _H, _D)


# speedup vs baseline: 1.2030x; 1.1635x over previous
import functools

import jax
import jax.numpy as jnp
from jax import lax
from jax.experimental import pallas as pl
from jax.experimental.pallas import tpu as pltpu
from jax.experimental.pallas import tpu_sc as plsc

_V = 1000000
_H = 20
_BT = 16384
_D = 64
_NC = 2
_NS = 16
_NW = _NC * _NS
_C = 256
_NB = _C // 128
_NCH = _BT // (_NW * _C)
_NT = _D // 8
_CP = 129
_FB = 8192

_mesh = plsc.VectorSubcoreMesh(core_axis_name="c", subcore_axis_name="s")

_DNUMS = lax.GatherDimensionNumbers(
    offset_dims=(), collapsed_slice_dims=(0,), start_index_map=(0,)
)


def _fmt_body(wt_ref, o_ref):
    for m in range(_FB // 256):
        a = wt_ref[:, pl.ds(m * 256, 128)].T
        b = wt_ref[:, pl.ds(m * 256 + 128, 128)].T
        o_ref[pl.ds(m * 128, 128), 0:_D] = a
        o_ref[pl.ds(m * 128, 128), _D : 2 * _D] = b


_NG = (_V + _FB - 1) // _FB
_VP = _NG * (_FB // 2)

_fmt = pl.pallas_call(
    _fmt_body,
    grid=(_NG,),
    in_specs=[pl.BlockSpec((_D, _FB), lambda i: (0, i))],
    out_specs=pl.BlockSpec((_FB // 2, 2 * _D), lambda i: (i, 0)),
    out_shape=jax.ShapeDtypeStruct((_VP, 2 * _D), jnp.float32),
)


@functools.partial(
    pl.kernel,
    out_type=jax.ShapeDtypeStruct((_H, _NT, _BT // 128, 8, 128), jnp.float32),
    mesh=_mesh,
    scratch_types=[
        pltpu.VMEM((2, _C), jnp.int32),
        pltpu.VMEM((2, _C), jnp.int32),
        pltpu.VMEM((2, _C, _D), jnp.float32),
        pltpu.VMEM((2, _NB, _NT, 8, _CP), jnp.float32),
        pltpu.SemaphoreType.DMA,
        pltpu.SemaphoreType.DMA,
    ],
    compiler_params=pltpu.CompilerParams(
        use_tc_tiling_on_sc=False, needs_layout_passes=False
    ),
)
def _emb_kernel(
    idx_hbm, table_hbm, out_hbm, idx_v, row_v, rows_v, out_v, gsem, osem
):
    wid = lax.axis_index("s") * _NC + lax.axis_index("c")
    lane = lax.iota(jnp.int32, 16)
    lane_t = lane // 8
    lane_r = lax.rem(lane, 8)
    ngch = _H * _NCH

    def chunk_off(g):
        h = g // _NCH
        sub = lax.rem(g, _NCH)
        return h * _BT + wid * (_NCH * _C) + sub * _C

    def idx_stage(g, slot):
        pltpu.sync_copy(idx_hbm.at[pl.ds(chunk_off(g), _C)], idx_v.at[slot])

        def split(k, _):
            v = idx_v[slot, pl.ds(k * 16, 16)]
            row_v[slot, pl.ds(k * 16, 16)] = (
                ((v >> 8) << 8) | ((v & 127) << 1) | ((v >> 7) & 1)
            )
            return 0

        lax.fori_loop(0, _C // 16, split, 0)

    def gather(slot):
        return pltpu.async_copy(
            table_hbm.at[row_v.at[slot]], rows_v.at[slot], gsem
        )

    def out_copy(g, t):
        h = g // _NCH
        sub = lax.rem(g, _NCH)
        slot = lax.rem(g, 2)
        blk0 = wid * (_NCH * _NB) + sub * _NB
        return pltpu.make_async_copy(
            out_v.at[slot, :, t, :, pl.ds(0, 128)],
            out_hbm.at[h, t, pl.ds(blk0, _NB), :, :],
            osem,
        )

    idx_stage(0, 0)
    gather(0)

    def chunk(g, _):
        slot = lax.rem(g, 2)
        nslot = 1 - slot

        @pl.when(g < ngch - 1)
        def _prefetch():
            idx_stage(g + 1, nslot)
            gather(nslot)

        pltpu.make_async_copy(
            table_hbm.at[row_v.at[slot]], rows_v.at[slot], gsem
        ).wait()

        @pl.when(g > 1)
        def _drain():
            for t in range(_NT):
                out_copy(g - 2, t).wait()

        slot_splat = jnp.full((16,), slot, jnp.int32)

        def b_body(bb, _):
            base_b = jnp.full((16,), bb, jnp.int32)

            def c_body(cq, _):
                for dk in range(4):
                    c = cq * 4 + dk
                    j = bb * 128 + c
                    base_c = jnp.full((16,), c, jnp.int32)
                    for d0 in range(0, _D, 16):
                        val = rows_v[slot, j, pl.ds(d0, 16)]
                        plsc.store_scatter(
                            out_v,
                            [slot_splat, base_b, lane_t + (d0 // 8), lane_r, base_c],
                            val,
                        )
                return 0

            lax.fori_loop(0, 32, c_body, 0)
            return 0

        lax.fori_loop(0, _NB, b_body, 0)

        for t in range(_NT):
            out_copy(g, t).start()
        return 0

    lax.fori_loop(0, ngch, chunk, 0)
    for t in range(_NT):
        out_copy(ngch - 2, t).wait()
        out_copy(ngch - 1, t).wait()


def kernel(indices, weight):
    idx_t = indices.astype(jnp.int32).T.reshape(-1)
    table = _fmt(weight.T).reshape(2 * _VP, _D)
    p5 = _emb_kernel(idx_t, table)
    return p5.transpose((2, 4, 0, 1, 3)).reshape(_BT, _H, _D)


# speedup vs baseline: 1.2950x; 1.0765x over previous
import functools

import jax
import jax.numpy as jnp
from jax import lax
from jax.experimental import pallas as pl
from jax.experimental.pallas import tpu as pltpu
from jax.experimental.pallas import tpu_sc as plsc

_V = 1000000
_H = 20
_BT = 16384
_D = 64
_NC = 2
_NS = 16
_NW = _NC * _NS
_C = 256
_NB = _C // 128
_NCH = _BT // (_NW * _C)
_NT = _D // 8
_CP = 129
_FB = 16384

_mesh = plsc.VectorSubcoreMesh(core_axis_name="c", subcore_axis_name="s")

_DNUMS = lax.GatherDimensionNumbers(
    offset_dims=(), collapsed_slice_dims=(0,), start_index_map=(0,)
)


def _fmt_body(wt_ref, o_ref):
    for m in range(_FB // 256):
        a = wt_ref[:, pl.ds(m * 256, 128)].T
        b = wt_ref[:, pl.ds(m * 256 + 128, 128)].T
        o_ref[pl.ds(m * 128, 128), 0:_D] = a
        o_ref[pl.ds(m * 128, 128), _D : 2 * _D] = b


_NG = (_V + _FB - 1) // _FB
_VP = _NG * (_FB // 2)

_fmt = pl.pallas_call(
    _fmt_body,
    grid=(_NG,),
    in_specs=[pl.BlockSpec((_D, _FB), lambda i: (0, i))],
    out_specs=pl.BlockSpec((_FB // 2, 2 * _D), lambda i: (i, 0)),
    out_shape=jax.ShapeDtypeStruct((_VP, 2 * _D), jnp.float32),
)


@functools.partial(
    pl.kernel,
    out_type=jax.ShapeDtypeStruct((_H, _NT, _BT // 128, 8, 128), jnp.float32),
    mesh=_mesh,
    scratch_types=[
        pltpu.VMEM((2, _C), jnp.int32),
        pltpu.VMEM((2, _C), jnp.int32),
        pltpu.VMEM((2, _C, _D), jnp.float32),
        pltpu.VMEM((2, _NB, _NT, 8, _CP), jnp.float32),
        pltpu.SemaphoreType.DMA,
        pltpu.SemaphoreType.DMA,
    ],
    compiler_params=pltpu.CompilerParams(
        use_tc_tiling_on_sc=False, needs_layout_passes=False
    ),
)
def _emb_kernel(
    idx_hbm, table_hbm, out_hbm, idx_v, row_v, rows_v, out_v, gsem, osem
):
    wid = lax.axis_index("s") * _NC + lax.axis_index("c")
    lane = lax.iota(jnp.int32, 16)
    lane_t = lane // 8
    lane_r = lax.rem(lane, 8)
    ngch = _H * _NCH

    def chunk_off(g):
        h = g // _NCH
        sub = lax.rem(g, _NCH)
        return h * _BT + wid * (_NCH * _C) + sub * _C

    def idx_stage(g, slot):
        pltpu.sync_copy(idx_hbm.at[pl.ds(chunk_off(g), _C)], idx_v.at[slot])

        def split(k, _):
            v = idx_v[slot, pl.ds(k * 16, 16)]
            row_v[slot, pl.ds(k * 16, 16)] = (
                ((v >> 8) << 8) | ((v & 127) << 1) | ((v >> 7) & 1)
            )
            return 0

        lax.fori_loop(0, _C // 16, split, 0)

    def gather(slot):
        return pltpu.async_copy(
            table_hbm.at[row_v.at[slot]], rows_v.at[slot], gsem
        )

    def out_copy(g, t):
        h = g // _NCH
        sub = lax.rem(g, _NCH)
        slot = lax.rem(g, 2)
        blk0 = wid * (_NCH * _NB) + sub * _NB
        return pltpu.make_async_copy(
            out_v.at[slot, :, t, :, pl.ds(0, 128)],
            out_hbm.at[h, t, pl.ds(blk0, _NB), :, :],
            osem,
        )

    idx_stage(0, 0)
    gather(0)

    def chunk(g, _):
        slot = lax.rem(g, 2)
        nslot = 1 - slot

        @pl.when(g < ngch - 1)
        def _prefetch():
            idx_stage(g + 1, nslot)
            gather(nslot)

        pltpu.make_async_copy(
            table_hbm.at[row_v.at[slot]], rows_v.at[slot], gsem
        ).wait()

        @pl.when(g > 1)
        def _drain():
            for t in range(_NT):
                out_copy(g - 2, t).wait()

        slot_splat = jnp.full((16,), slot, jnp.int32)

        def b_body(bb, _):
            base_b = jnp.full((16,), bb, jnp.int32)

            def c_body(cq, _):
                for dk in range(8):
                    c = cq * 8 + dk
                    j = bb * 128 + c
                    base_c = jnp.full((16,), c, jnp.int32)
                    for d0 in range(0, _D, 16):
                        val = rows_v[slot, j, pl.ds(d0, 16)]
                        plsc.store_scatter(
                            out_v,
                            [slot_splat, base_b, lane_t + (d0 // 8), lane_r, base_c],
                            val,
                        )
                return 0

            lax.fori_loop(0, 16, c_body, 0)
            return 0

        lax.fori_loop(0, _NB, b_body, 0)

        for t in range(_NT):
            out_copy(g, t).start()
        return 0

    lax.fori_loop(0, ngch, chunk, 0)
    for t in range(_NT):
        out_copy(ngch - 2, t).wait()
        out_copy(ngch - 1, t).wait()


def kernel(indices, weight):
    idx_t = indices.astype(jnp.int32).T.reshape(-1)
    table = _fmt(weight.T).reshape(2 * _VP, _D)
    p5 = _emb_kernel(idx_t, table)
    return p5.transpose((2, 4, 0, 1, 3)).reshape(_BT, _H, _D)


# speedup vs baseline: 1.3412x; 1.0357x over previous
import functools

import jax
import jax.numpy as jnp
from jax import lax
from jax.experimental import pallas as pl
from jax.experimental.pallas import tpu as pltpu
from jax.experimental.pallas import tpu_sc as plsc

_V = 1000000
_H = 20
_BT = 16384
_D = 64
_NC = 2
_NS = 16
_NW = _NC * _NS
_C = 256
_NB = _C // 128
_NCH = _BT // (_NW * _C)
_NT = _D // 8
_CP = 129
_FB = 32768

_mesh = plsc.VectorSubcoreMesh(core_axis_name="c", subcore_axis_name="s")

_DNUMS = lax.GatherDimensionNumbers(
    offset_dims=(), collapsed_slice_dims=(0,), start_index_map=(0,)
)


def _fmt_body(wt_ref, o_ref):
    for m in range(_FB // 256):
        a = wt_ref[:, pl.ds(m * 256, 128)].T
        b = wt_ref[:, pl.ds(m * 256 + 128, 128)].T
        o_ref[pl.ds(m * 128, 128), 0:_D] = a
        o_ref[pl.ds(m * 128, 128), _D : 2 * _D] = b


_NG = (_V + _FB - 1) // _FB
_VP = _NG * (_FB // 2)

_fmt = pl.pallas_call(
    _fmt_body,
    grid=(_NG,),
    in_specs=[pl.BlockSpec((_D, _FB), lambda i: (0, i))],
    out_specs=pl.BlockSpec((_FB // 2, 2 * _D), lambda i: (i, 0)),
    out_shape=jax.ShapeDtypeStruct((_VP, 2 * _D), jnp.float32),
)


@functools.partial(
    pl.kernel,
    out_type=jax.ShapeDtypeStruct((_H, _NT, _BT // 128, 8, 128), jnp.float32),
    mesh=_mesh,
    scratch_types=[
        pltpu.VMEM((2, _C), jnp.int32),
        pltpu.VMEM((2, _C), jnp.int32),
        pltpu.VMEM((2, _C, _D), jnp.float32),
        pltpu.VMEM((2, _NB, _NT, 8, _CP), jnp.float32),
        pltpu.SemaphoreType.DMA,
        pltpu.SemaphoreType.DMA,
    ],
    compiler_params=pltpu.CompilerParams(
        use_tc_tiling_on_sc=False, needs_layout_passes=False
    ),
)
def _emb_kernel(
    idx_hbm, table_hbm, out_hbm, idx_v, row_v, rows_v, out_v, gsem, osem
):
    wid = lax.axis_index("s") * _NC + lax.axis_index("c")
    lane = lax.iota(jnp.int32, 16)
    lane_t = lane // 8
    lane_r = lax.rem(lane, 8)
    ngch = _H * _NCH

    def chunk_off(g):
        h = g // _NCH
        sub = lax.rem(g, _NCH)
        return h * _BT + wid * (_NCH * _C) + sub * _C

    def idx_stage(g, slot):
        pltpu.sync_copy(idx_hbm.at[pl.ds(chunk_off(g), _C)], idx_v.at[slot])

        def split(k, _):
            v = idx_v[slot, pl.ds(k * 16, 16)]
            row_v[slot, pl.ds(k * 16, 16)] = (
                ((v >> 8) << 8) | ((v & 127) << 1) | ((v >> 7) & 1)
            )
            return 0

        lax.fori_loop(0, _C // 16, split, 0)

    def gather(slot):
        return pltpu.async_copy(
            table_hbm.at[row_v.at[slot]], rows_v.at[slot], gsem
        )

    def out_copy(g, t):
        h = g // _NCH
        sub = lax.rem(g, _NCH)
        slot = lax.rem(g, 2)
        blk0 = wid * (_NCH * _NB) + sub * _NB
        return pltpu.make_async_copy(
            out_v.at[slot, :, t, :, pl.ds(0, 128)],
            out_hbm.at[h, t, pl.ds(blk0, _NB), :, :],
            osem,
        )

    idx_stage(0, 0)
    gather(0)

    def chunk(g, _):
        slot = lax.rem(g, 2)
        nslot = 1 - slot

        @pl.when(g < ngch - 1)
        def _prefetch():
            idx_stage(g + 1, nslot)
            gather(nslot)

        pltpu.make_async_copy(
            table_hbm.at[row_v.at[slot]], rows_v.at[slot], gsem
        ).wait()

        @pl.when(g > 1)
        def _drain():
            for t in range(_NT):
                out_copy(g - 2, t).wait()

        slot_splat = jnp.full((16,), slot, jnp.int32)

        def b_body(bb, _):
            base_b = jnp.full((16,), bb, jnp.int32)

            def c_body(cq, _):
                for dk in range(8):
                    c = cq * 8 + dk
                    j = bb * 128 + c
                    base_c = jnp.full((16,), c, jnp.int32)
                    for d0 in range(0, _D, 16):
                        val = rows_v[slot, j, pl.ds(d0, 16)]
                        plsc.store_scatter(
                            out_v,
                            [slot_splat, base_b, lane_t + (d0 // 8), lane_r, base_c],
                            val,
                        )
                return 0

            lax.fori_loop(0, 16, c_body, 0)
            return 0

        lax.fori_loop(0, _NB, b_body, 0)

        for t in range(_NT):
            out_copy(g, t).start()
        return 0

    lax.fori_loop(0, ngch, chunk, 0)
    for t in range(_NT):
        out_copy(ngch - 2, t).wait()
        out_copy(ngch - 1, t).wait()


def kernel(indices, weight):
    idx_t = indices.astype(jnp.int32).T.reshape(-1)
    table = _fmt(weight.T).reshape(2 * _VP, _D)
    p5 = _emb_kernel(idx_t, table)
    return p5.transpose((2, 4, 0, 1, 3)).reshape(_BT, _H, _D)


# speedup vs baseline: 1.3421x; 1.0007x over previous
import functools

import jax
import jax.numpy as jnp
from jax import lax
from jax.experimental import pallas as pl
from jax.experimental.pallas import tpu as pltpu
from jax.experimental.pallas import tpu_sc as plsc

_V = 1000000
_H = 20
_BT = 16384
_D = 64
_NC = 2
_NS = 16
_NW = _NC * _NS
_C = 256
_NB = _C // 128
_NCH = _BT // (_NW * _C)
_NT = _D // 8
_CP = 129
_FB = 32768

_mesh = plsc.VectorSubcoreMesh(core_axis_name="c", subcore_axis_name="s")

_DNUMS = lax.GatherDimensionNumbers(
    offset_dims=(), collapsed_slice_dims=(0,), start_index_map=(0,)
)


def _fmt_body(wt_ref, o_ref):
    for m in range(_FB // 256):
        a = wt_ref[:, pl.ds(m * 256, 128)].T
        b = wt_ref[:, pl.ds(m * 256 + 128, 128)].T
        o_ref[pl.ds(m * 128, 128), :] = jnp.concatenate([a, b], axis=1)


_NG = (_V + _FB - 1) // _FB
_VP = _NG * (_FB // 2)

_fmt = pl.pallas_call(
    _fmt_body,
    grid=(_NG,),
    in_specs=[pl.BlockSpec((_D, _FB), lambda i: (0, i))],
    out_specs=pl.BlockSpec((_FB // 2, 2 * _D), lambda i: (i, 0)),
    out_shape=jax.ShapeDtypeStruct((_VP, 2 * _D), jnp.float32),
)


@functools.partial(
    pl.kernel,
    out_type=jax.ShapeDtypeStruct((_H, _NT, _BT // 128, 8, 128), jnp.float32),
    mesh=_mesh,
    scratch_types=[
        pltpu.VMEM((2, _C), jnp.int32),
        pltpu.VMEM((2, _C), jnp.int32),
        pltpu.VMEM((2, _C, _D), jnp.float32),
        pltpu.VMEM((2, _NB, _NT, 8, _CP), jnp.float32),
        pltpu.SemaphoreType.DMA,
        pltpu.SemaphoreType.DMA,
    ],
    compiler_params=pltpu.CompilerParams(
        use_tc_tiling_on_sc=False, needs_layout_passes=False
    ),
)
def _emb_kernel(
    idx_hbm, table_hbm, out_hbm, idx_v, row_v, rows_v, out_v, gsem, osem
):
    wid = lax.axis_index("s") * _NC + lax.axis_index("c")
    lane = lax.iota(jnp.int32, 16)
    lane_t = lane // 8
    lane_r = lax.rem(lane, 8)
    ngch = _H * _NCH

    def chunk_off(g):
        h = g // _NCH
        sub = lax.rem(g, _NCH)
        return h * _BT + wid * (_NCH * _C) + sub * _C

    def idx_stage(g, slot):
        pltpu.sync_copy(idx_hbm.at[pl.ds(chunk_off(g), _C)], idx_v.at[slot])

        def split(k, _):
            v = idx_v[slot, pl.ds(k * 16, 16)]
            row_v[slot, pl.ds(k * 16, 16)] = (
                ((v >> 8) << 8) | ((v & 127) << 1) | ((v >> 7) & 1)
            )
            return 0

        lax.fori_loop(0, _C // 16, split, 0)

    def gather(slot):
        return pltpu.async_copy(
            table_hbm.at[row_v.at[slot]], rows_v.at[slot], gsem
        )

    def out_copy(g, t):
        h = g // _NCH
        sub = lax.rem(g, _NCH)
        slot = lax.rem(g, 2)
        blk0 = wid * (_NCH * _NB) + sub * _NB
        return pltpu.make_async_copy(
            out_v.at[slot, :, t, :, pl.ds(0, 128)],
            out_hbm.at[h, t, pl.ds(blk0, _NB), :, :],
            osem,
        )

    idx_stage(0, 0)
    gather(0)

    def chunk(g, _):
        slot = lax.rem(g, 2)
        nslot = 1 - slot

        @pl.when(g < ngch - 1)
        def _prefetch():
            idx_stage(g + 1, nslot)
            gather(nslot)

        pltpu.make_async_copy(
            table_hbm.at[row_v.at[slot]], rows_v.at[slot], gsem
        ).wait()

        @pl.when(g > 1)
        def _drain():
            for t in range(_NT):
                out_copy(g - 2, t).wait()

        slot_splat = jnp.full((16,), slot, jnp.int32)

        def b_body(bb, _):
            base_b = jnp.full((16,), bb, jnp.int32)

            def c_body(cq, _):
                for dk in range(8):
                    c = cq * 8 + dk
                    j = bb * 128 + c
                    base_c = jnp.full((16,), c, jnp.int32)
                    for d0 in range(0, _D, 16):
                        val = rows_v[slot, j, pl.ds(d0, 16)]
                        plsc.store_scatter(
                            out_v,
                            [slot_splat, base_b, lane_t + (d0 // 8), lane_r, base_c],
                            val,
                        )
                return 0

            lax.fori_loop(0, 16, c_body, 0)
            return 0

        lax.fori_loop(0, _NB, b_body, 0)

        for t in range(_NT):
            out_copy(g, t).start()
        return 0

    lax.fori_loop(0, ngch, chunk, 0)
    for t in range(_NT):
        out_copy(ngch - 2, t).wait()
        out_copy(ngch - 1, t).wait()


def kernel(indices, weight):
    idx_t = indices.astype(jnp.int32).T.reshape(-1)
    table = _fmt(weight.T).reshape(2 * _VP, _D)
    p5 = _emb_kernel(idx_t, table)
    return p5.transpose((2, 4, 0, 1, 3)).reshape(_BT, _H, _D)


# speedup vs baseline: 1.4405x; 1.0733x over previous
import functools

import jax
import jax.numpy as jnp
from jax import lax
from jax.experimental import pallas as pl
from jax.experimental.pallas import tpu as pltpu
from jax.experimental.pallas import tpu_sc as plsc

_V = 1000000
_H = 20
_BT = 16384
_D = 64
_NC = 2
_NS = 16
_NW = _NC * _NS
_C = 256
_NB = _C // 128
_NCH = _BT // (_NW * _C)
_NT = _D // 8
_CP = 129
_FB = 32768

_mesh = plsc.VectorSubcoreMesh(core_axis_name="c", subcore_axis_name="s")

_DNUMS = lax.GatherDimensionNumbers(
    offset_dims=(), collapsed_slice_dims=(0,), start_index_map=(0,)
)


def _fmt_body(wt_ref, o_ref):
    for m in range(_FB // 256):
        a = wt_ref[:, pl.ds(m * 256, 128)].T
        b = wt_ref[:, pl.ds(m * 256 + 128, 128)].T
        o_ref[pl.ds(m * 128, 128), :] = jnp.concatenate([a, b], axis=1)


_NG = (_V + _FB - 1) // _FB
_VP = _NG * (_FB // 2)

_fmt = pl.pallas_call(
    _fmt_body,
    grid=(_NG,),
    in_specs=[pl.BlockSpec((_D, _FB), lambda i: (0, i))],
    out_specs=pl.BlockSpec((_FB // 2, 2 * _D), lambda i: (i, 0)),
    out_shape=jax.ShapeDtypeStruct((_VP, 2 * _D), jnp.float32),
)


@functools.partial(
    pl.kernel,
    out_type=jax.ShapeDtypeStruct((_H, _NT, _BT // 128, 8, 128), jnp.float32),
    mesh=_mesh,
    scratch_types=[
        pltpu.VMEM((_H, _NCH * _C), jnp.int32),
        pltpu.VMEM((_H, _NCH * _C), jnp.int32),
        pltpu.VMEM((2, _C, _D), jnp.float32),
        pltpu.VMEM((2, _NB, _NT, 8, _CP), jnp.float32),
        pltpu.SemaphoreType.DMA,
        pltpu.SemaphoreType.DMA,
    ],
    compiler_params=pltpu.CompilerParams(
        use_tc_tiling_on_sc=False, needs_layout_passes=False
    ),
)
def _emb_kernel(
    idx_hbm, table_hbm, out_hbm, idx_v, row_v, rows_v, out_v, gsem, osem
):
    wid = lax.axis_index("s") * _NC + lax.axis_index("c")
    lane = lax.iota(jnp.int32, 16)
    lane_t = lane // 8
    lane_r = lax.rem(lane, 8)
    ngch = _H * _NCH

    pltpu.sync_copy(
        idx_hbm.at[:, pl.ds(wid * (_NCH * _C), _NCH * _C)], idx_v
    )

    def _split(i, _):
        h = i // (_NCH * _C // 16)
        k = lax.rem(i, _NCH * _C // 16)
        v = idx_v[h, pl.ds(k * 16, 16)]
        row_v[h, pl.ds(k * 16, 16)] = (
            ((v >> 8) << 8) | ((v & 127) << 1) | ((v >> 7) & 1)
        )
        return 0

    lax.fori_loop(0, _H * (_NCH * _C // 16), _split, 0)

    def gather(g, slot):
        h = g // _NCH
        sub = lax.rem(g, _NCH)
        return pltpu.async_copy(
            table_hbm.at[row_v.at[h, pl.ds(sub * _C, _C)]],
            rows_v.at[slot],
            gsem,
        )

    def out_copy(g, t):
        h = g // _NCH
        sub = lax.rem(g, _NCH)
        slot = lax.rem(g, 2)
        blk0 = wid * (_NCH * _NB) + sub * _NB
        return pltpu.make_async_copy(
            out_v.at[slot, :, t, :, pl.ds(0, 128)],
            out_hbm.at[h, t, pl.ds(blk0, _NB), :, :],
            osem,
        )

    gather(0, 0)

    def chunk(g, _):
        slot = lax.rem(g, 2)
        nslot = 1 - slot

        @pl.when(g < ngch - 1)
        def _prefetch():
            gather(g + 1, nslot)

        gather_wait = g
        h = gather_wait // _NCH
        sub = lax.rem(gather_wait, _NCH)
        pltpu.make_async_copy(
            table_hbm.at[row_v.at[h, pl.ds(sub * _C, _C)]],
            rows_v.at[slot],
            gsem,
        ).wait()

        @pl.when(g > 1)
        def _drain():
            for t in range(_NT):
                out_copy(g - 2, t).wait()

        slot_splat = jnp.full((16,), slot, jnp.int32)

        def b_body(bb, _):
            base_b = jnp.full((16,), bb, jnp.int32)

            def c_body(cq, _):
                for dk in range(8):
                    c = cq * 8 + dk
                    j = bb * 128 + c
                    base_c = jnp.full((16,), c, jnp.int32)
                    for d0 in range(0, _D, 16):
                        val = rows_v[slot, j, pl.ds(d0, 16)]
                        plsc.store_scatter(
                            out_v,
                            [slot_splat, base_b, lane_t + (d0 // 8), lane_r, base_c],
                            val,
                        )
                return 0

            lax.fori_loop(0, 16, c_body, 0)
            return 0

        lax.fori_loop(0, _NB, b_body, 0)

        for t in range(_NT):
            out_copy(g, t).start()
        return 0

    lax.fori_loop(0, ngch, chunk, 0)
    for t in range(_NT):
        out_copy(ngch - 2, t).wait()
        out_copy(ngch - 1, t).wait()


def kernel(indices, weight):
    idx_t = indices.astype(jnp.int32).T
    table = _fmt(weight.T).reshape(2 * _VP, _D)
    p5 = _emb_kernel(idx_t, table)
    return p5.transpose((2, 4, 0, 1, 3)).reshape(_BT, _H, _D)


# speedup vs baseline: 1.4414x; 1.0006x over previous
import functools

import jax
import jax.numpy as jnp
from jax import lax
from jax.experimental import pallas as pl
from jax.experimental.pallas import tpu as pltpu
from jax.experimental.pallas import tpu_sc as plsc

_V = 1000000
_H = 20
_BT = 16384
_D = 64
_NC = 2
_NS = 16
_NW = _NC * _NS
_C = 256
_NB = _C // 128
_NCH = _BT // (_NW * _C)
_NT = _D // 8
_CP = 129
_FB = 32768

_mesh = plsc.VectorSubcoreMesh(core_axis_name="c", subcore_axis_name="s")


def _fmt_body(wt_ref, o_ref):
    for m in range(_FB // 256):
        a = wt_ref[:, pl.ds(m * 256, 128)].T
        b = wt_ref[:, pl.ds(m * 256 + 128, 128)].T
        o_ref[pl.ds(m * 128, 128), :] = jnp.concatenate([a, b], axis=1)


_NG = (_V + _FB - 1) // _FB
_VP = _NG * (_FB // 2)

_fmt = pl.pallas_call(
    _fmt_body,
    grid=(_NG,),
    in_specs=[pl.BlockSpec((_D, _FB), lambda i: (0, i))],
    out_specs=pl.BlockSpec((_FB // 2, 2 * _D), lambda i: (i, 0)),
    out_shape=jax.ShapeDtypeStruct((_VP, 2 * _D), jnp.float32),
)


@functools.partial(
    pl.kernel,
    out_type=jax.ShapeDtypeStruct((_H, _NT, _BT // 128, 8, 128), jnp.float32),
    mesh=_mesh,
    scratch_types=[
        pltpu.VMEM((_H, _NCH * _C), jnp.int32),
        pltpu.VMEM((_H, _NCH * _C), jnp.int32),
        pltpu.VMEM((2, _C, _D), jnp.float32),
        pltpu.VMEM((2, _NB, _NT, 8, _CP), jnp.float32),
        pltpu.SemaphoreType.DMA,
        pltpu.SemaphoreType.DMA,
    ],
    compiler_params=pltpu.CompilerParams(
        use_tc_tiling_on_sc=False, needs_layout_passes=False
    ),
)
def _emb_kernel(
    idx_hbm, table_hbm, out_hbm, idx_v, row_v, rows_v, out_v, gsem, osem
):
    wid = lax.axis_index("s") * _NC + lax.axis_index("c")
    lane = lax.iota(jnp.int32, 16)
    lane_t = lane // 8
    lane_r = lax.rem(lane, 8)
    ngch = _H * _NCH

    pltpu.sync_copy(
        idx_hbm.at[:, pl.ds(wid * (_NCH * _C), _NCH * _C)], idx_v
    )

    def _split(i, _):
        h = i // (_NCH * _C // 16)
        k = lax.rem(i, _NCH * _C // 16)
        v = idx_v[h, pl.ds(k * 16, 16)]
        row_v[h, pl.ds(k * 16, 16)] = (
            ((v >> 8) << 8) | ((v & 127) << 1) | ((v >> 7) & 1)
        )
        return 0

    lax.fori_loop(0, _H * (_NCH * _C // 16), _split, 0)

    def gather(g, slot):
        h = g // _NCH
        sub = lax.rem(g, _NCH)
        return pltpu.async_copy(
            table_hbm.at[row_v.at[h, pl.ds(sub * _C, _C)]],
            rows_v.at[slot],
            gsem,
        )

    def out_copy(g, t):
        h = g // _NCH
        sub = lax.rem(g, _NCH)
        slot = lax.rem(g, 2)
        blk0 = wid * (_NCH * _NB) + sub * _NB
        return pltpu.make_async_copy(
            out_v.at[slot, :, t, :, pl.ds(0, 128)],
            out_hbm.at[h, t, pl.ds(blk0, _NB), :, :],
            osem,
        )

    gather(0, 0)

    def chunk(g, _):
        slot = lax.rem(g, 2)
        nslot = 1 - slot

        @pl.when(g < ngch - 1)
        def _prefetch():
            gather(g + 1, nslot)

        gather_wait = g
        h = gather_wait // _NCH
        sub = lax.rem(gather_wait, _NCH)
        pltpu.make_async_copy(
            table_hbm.at[row_v.at[h, pl.ds(sub * _C, _C)]],
            rows_v.at[slot],
            gsem,
        ).wait()

        @pl.when(g > 1)
        def _drain():
            for t in range(_NT):
                out_copy(g - 2, t).wait()

        slot_splat = jnp.full((16,), slot, jnp.int32)

        def b_body(bb, _):
            base_b = jnp.full((16,), bb, jnp.int32)

            def c_body(cq, _):
                for dk in range(8):
                    c = cq * 8 + dk
                    j = bb * 128 + c
                    base_c = jnp.full((16,), c, jnp.int32)
                    for d0 in range(0, _D, 16):
                        val = rows_v[slot, j, pl.ds(d0, 16)]
                        plsc.store_scatter(
                            out_v,
                            [slot_splat, base_b, lane_t + (d0 // 8), lane_r, base_c],
                            val,
                        )
                return 0

            lax.fori_loop(0, 16, c_body, 0)
            return 0

        lax.fori_loop(0, _NB, b_body, 0)

        for t in range(_NT):
            out_copy(g, t).start()
        return 0

    lax.fori_loop(0, ngch, chunk, 0)
    for t in range(_NT):
        out_copy(ngch - 2, t).wait()
        out_copy(ngch - 1, t).wait()


def kernel(indices, weight):
    idx_t = indices.astype(jnp.int32).T
    table = _fmt(weight.T).reshape(2 * _VP, _D)
    p5 = _emb_kernel(idx_t, table)
    return p5.transpose((2, 4, 0, 1, 3)).reshape(_BT, _H, _D)
